# baseline (speedup 1.0000x reference)
import functools
import jax
import jax.numpy as jnp
from jax import lax
from jax.experimental import pallas as pl
from jax.experimental.pallas import tpu as pltpu
from jax.experimental.pallas import tpu_sc as plsc

_B, _S = 2, 2048
_D = 1024
_H = 2048
_E = 8
_N = _B * _S
_TT = 256
_TPE = _N // _E
_NI = _TPE // _TT

_NC, _NS = 2, 16
_NW = _NC * _NS
_RPW = 2 * _N // _NW
_CH = 32
_NCH = _RPW // _CH


def _router_body(x_ref, wr_ref, probs_ref, gidx_ref, gates_ref):
    x = x_ref[...]
    logits = lax.dot_general(x, wr_ref[...], (((1,), (1,)), ((), ())),
                             preferred_element_type=jnp.float32)
    p1 = jax.nn.softmax(logits, axis=-1)
    probs = jax.nn.softmax(p1, axis=-1)
    lane = lax.broadcasted_iota(jnp.int32, (_N, _E), 1)
    m1 = jnp.max(probs, axis=-1, keepdims=True)
    i1 = jnp.min(jnp.where(probs == m1, lane, _E), axis=-1, keepdims=True)
    p_wo = jnp.where(lane == i1, -1.0, probs)
    m2 = jnp.max(p_wo, axis=-1, keepdims=True)
    i2 = jnp.min(jnp.where(p_wo == m2, lane, _E), axis=-1, keepdims=True)
    row = lax.broadcasted_iota(jnp.int32, (_N, 1), 0)
    base8 = 8 * (row % _TPE)
    probs_ref[...] = probs
    gidx_ref[:, 0:1] = base8 + i1
    gidx_ref[:, 1:2] = base8 + i2
    gates_ref[:, 0:1] = m1
    gates_ref[:, 1:2] = m2


def _router_call(x, Wr):
    return pl.pallas_call(
        _router_body,
        in_specs=[
            pl.BlockSpec((_N, _D), lambda: (0, 0)),
            pl.BlockSpec((_E, _D), lambda: (0, 0)),
        ],
        out_specs=[
            pl.BlockSpec((_N, _E), lambda: (0, 0)),
            pl.BlockSpec((_N, 2), lambda: (0, 0)),
            pl.BlockSpec((_N, 2), lambda: (0, 0)),
        ],
        out_shape=[
            jax.ShapeDtypeStruct((_N, _E), jnp.float32),
            jax.ShapeDtypeStruct((_N, 2), jnp.int32),
            jax.ShapeDtypeStruct((_N, 2), jnp.float32),
        ],
    )(x, Wr)


def _sc_gather(x, gidx_flat):
    mesh = plsc.VectorSubcoreMesh(core_axis_name="c", subcore_axis_name="s")
    row_w = x.shape[1]
    n_rows = gidx_flat.shape[0]
    rpw = n_rows // _NW
    nch = rpw // _CH

    @functools.partial(
        pl.kernel,
        mesh=mesh,
        out_type=jax.ShapeDtypeStruct((n_rows, row_w), x.dtype),
        scratch_types=[
            pltpu.VMEM((nch, _CH), jnp.int32),
            pltpu.VMEM((_CH, row_w), x.dtype),
            pltpu.VMEM((_CH, row_w), x.dtype),
            pltpu.SemaphoreType.DMA,
            pltpu.SemaphoreType.DMA,
        ],
    )
    def k(x_hbm, gidx_hbm, xs_hbm, idx_v, rows_a, rows_b, sem_a, sem_b):
        wid = lax.axis_index("s") * _NC + lax.axis_index("c")
        base = wid * rpw
        pltpu.sync_copy(gidx_hbm.at[wid], idx_v)
        bufs = (rows_a, rows_b)
        sems = (sem_a, sem_b)
        handles = [None, None]
        for c in range(nch):
            handles[c % 2] = pltpu.async_copy(
                x_hbm.at[idx_v.at[c]], bufs[c % 2], sems[c % 2])
            if c >= 1:
                handles[(c - 1) % 2].wait()
                pltpu.sync_copy(bufs[(c - 1) % 2],
                                xs_hbm.at[pl.ds(base + (c - 1) * _CH, _CH)])
        handles[(nch - 1) % 2].wait()
        pltpu.sync_copy(bufs[(nch - 1) % 2],
                        xs_hbm.at[pl.ds(base + (nch - 1) * _CH, _CH)])

    return k(x, gidx_flat.reshape(_NW, nch, _CH))


_HC = 512
_NHT = _H // _HC


def _expert_body(e0, xs0_ref, xs1_ref, w1x_ref, w1g_ref, w2_ref, gates_ref,
                 out_ref, xbf_scr):
    e = pl.program_id(0)
    ht = pl.program_id(1)

    @pl.when(ht == 0)
    def _stage_x():
        xbf_scr[:_TPE, :] = xs0_ref[0].astype(jnp.bfloat16)
        xbf_scr[_TPE:, :] = xs1_ref[0].astype(jnp.bfloat16)

    xx = xbf_scr[...]
    hx = lax.dot_general(xx, w1x_ref[0].astype(jnp.bfloat16),
                         (((1,), (1,)), ((), ())),
                         preferred_element_type=jnp.float32)
    hg = lax.dot_general(xx, w1g_ref[0].astype(jnp.bfloat16),
                         (((1,), (1,)), ((), ())),
                         preferred_element_type=jnp.float32)
    act = (0.5 * hg * (1.0 + lax.erf(hg * 0.7071067811865476))) * hx
    rows0 = e * _TPE
    gcols = gates_ref[pl.ds((e0 + e) * _TPE, _TPE), :]
    acomb = act[:_TPE] * gcols[:, 0:1] + act[_TPE:] * gcols[:, 1:2]
    contrib = lax.dot_general(acomb.astype(jnp.bfloat16),
                              w2_ref[0].astype(jnp.bfloat16),
                              (((1,), (1,)), ((), ())),
                              preferred_element_type=jnp.float32)

    @pl.when(ht == 0)
    def _init():
        out_ref[pl.ds(rows0, _TPE), :] = contrib

    @pl.when(ht != 0)
    def _acc():
        out_ref[pl.ds(rows0, _TPE), :] += contrib


def _expert_call(xs, W1, W2, gates, e0, ne):
    nt = ne * _TPE
    return pl.pallas_call(
        functools.partial(_expert_body, e0),
        grid=(ne, _NHT),
        in_specs=[
            pl.BlockSpec((1, _TPE, _D), lambda e, ht: (0, e, 0)),
            pl.BlockSpec((1, _TPE, _D), lambda e, ht: (1, e, 0)),
            pl.BlockSpec((1, _HC, _D), lambda e, ht: (e0 + e, ht, 0)),
            pl.BlockSpec((1, _HC, _D), lambda e, ht: (e0 + e, _NHT + ht, 0)),
            pl.BlockSpec((1, _D, _HC), lambda e, ht: (e0 + e, 0, ht)),
            pl.BlockSpec((_N, 2), lambda e, ht: (0, 0)),
        ],
        out_specs=pl.BlockSpec((nt, _D), lambda e, ht: (0, 0)),
        out_shape=jax.ShapeDtypeStruct((nt, _D), jnp.float32),
        scratch_shapes=[pltpu.VMEM((2 * _TPE, _D), jnp.bfloat16)],
    )(xs, xs, W1, W1, W2, gates)


_NSL = 2
_EPS = _E // _NSL
_TSL = _EPS * _TPE


@jax.jit
def kernel(inputs, W1, W2, Wr):
    x = inputs.reshape(_N, _D)
    probs, gidx, gates = _router_call(x, Wr)
    gidxT = gidx.T
    outs = []
    for sl in range(_NSL):
        gsl = gidxT[:, sl * _TSL:(sl + 1) * _TSL].reshape(2 * _TSL)
        xsl = _sc_gather(x, gsl).reshape(2, _TSL, _D)
        outs.append(_expert_call(xsl, W1, W2, gates, sl * _EPS, _EPS))
    out = jnp.concatenate(outs, axis=0)
    return out.reshape(_B, _S, _D), probs.reshape(_B, _S, _E)


# speedup vs baseline: 1.0603x; 1.0603x over previous
import functools
import jax
import jax.numpy as jnp
from jax import lax
from jax.experimental import pallas as pl
from jax.experimental.pallas import tpu as pltpu
from jax.experimental.pallas import tpu_sc as plsc

_B, _S = 2, 2048
_D = 1024
_H = 2048
_E = 8
_N = _B * _S
_TT = 256
_TPE = _N // _E
_NI = _TPE // _TT

_NC, _NS = 2, 16
_NW = _NC * _NS
_RPW = 2 * _N // _NW
_CH = 32
_NCH = _RPW // _CH


def _router_body(x_ref, wr_ref, probs_ref, gidx_ref, gates_ref):
    x = x_ref[...]
    logits = lax.dot_general(x, wr_ref[...], (((1,), (1,)), ((), ())),
                             preferred_element_type=jnp.float32)
    p1 = jax.nn.softmax(logits, axis=-1)
    probs = jax.nn.softmax(p1, axis=-1)
    lane = lax.broadcasted_iota(jnp.int32, (_N, _E), 1)
    m1 = jnp.max(probs, axis=-1, keepdims=True)
    i1 = jnp.min(jnp.where(probs == m1, lane, _E), axis=-1, keepdims=True)
    p_wo = jnp.where(lane == i1, -1.0, probs)
    m2 = jnp.max(p_wo, axis=-1, keepdims=True)
    i2 = jnp.min(jnp.where(p_wo == m2, lane, _E), axis=-1, keepdims=True)
    row = lax.broadcasted_iota(jnp.int32, (_N, 1), 0)
    base8 = 8 * (row % _TPE)
    probs_ref[...] = probs
    gidx_ref[:, 0:1] = base8 + i1
    gidx_ref[:, 1:2] = base8 + i2
    gates_ref[:, 0:1] = m1
    gates_ref[:, 1:2] = m2


def _router_call(x, Wr):
    return pl.pallas_call(
        _router_body,
        in_specs=[
            pl.BlockSpec((_N, _D), lambda: (0, 0)),
            pl.BlockSpec((_E, _D), lambda: (0, 0)),
        ],
        out_specs=[
            pl.BlockSpec((_N, _E), lambda: (0, 0)),
            pl.BlockSpec((_N, 2), lambda: (0, 0)),
            pl.BlockSpec((_N, 2), lambda: (0, 0)),
        ],
        out_shape=[
            jax.ShapeDtypeStruct((_N, _E), jnp.float32),
            jax.ShapeDtypeStruct((_N, 2), jnp.int32),
            jax.ShapeDtypeStruct((_N, 2), jnp.float32),
        ],
    )(x, Wr)


def _sc_gather(x, gidx_flat):
    mesh = plsc.VectorSubcoreMesh(core_axis_name="c", subcore_axis_name="s")
    row_w = x.shape[1]
    n_rows = gidx_flat.shape[0]
    rpw = n_rows // _NW
    nch = rpw // _CH

    @functools.partial(
        pl.kernel,
        mesh=mesh,
        out_type=jax.ShapeDtypeStruct((n_rows, row_w), x.dtype),
        scratch_types=[
            pltpu.VMEM((nch, _CH), jnp.int32),
            pltpu.VMEM((_CH, row_w), x.dtype),
            pltpu.VMEM((_CH, row_w), x.dtype),
            pltpu.SemaphoreType.DMA,
            pltpu.SemaphoreType.DMA,
        ],
    )
    def k(x_hbm, gidx_hbm, xs_hbm, idx_v, rows_a, rows_b, sem_a, sem_b):
        wid = lax.axis_index("s") * _NC + lax.axis_index("c")
        base = wid * rpw
        pltpu.sync_copy(gidx_hbm.at[wid], idx_v)
        bufs = (rows_a, rows_b)
        sems = (sem_a, sem_b)
        handles = [None, None]
        for c in range(nch):
            handles[c % 2] = pltpu.async_copy(
                x_hbm.at[idx_v.at[c]], bufs[c % 2], sems[c % 2])
            if c >= 1:
                handles[(c - 1) % 2].wait()
                pltpu.sync_copy(bufs[(c - 1) % 2],
                                xs_hbm.at[pl.ds(base + (c - 1) * _CH, _CH)])
        handles[(nch - 1) % 2].wait()
        pltpu.sync_copy(bufs[(nch - 1) % 2],
                        xs_hbm.at[pl.ds(base + (nch - 1) * _CH, _CH)])

    return k(x, gidx_flat.reshape(_NW, nch, _CH))


_HC = 512
_NHT = _H // _HC


def _expert_body(e0, xs0_ref, xs1_ref, w1x_ref, w1g_ref, w2_ref, gates_ref,
                 out_ref, xbf_scr):
    e = pl.program_id(0)
    ht = pl.program_id(1)

    @pl.when(ht == 0)
    def _stage_x():
        xbf_scr[:_TPE, :] = xs0_ref[0].astype(jnp.bfloat16)
        xbf_scr[_TPE:, :] = xs1_ref[0].astype(jnp.bfloat16)

    xx = xbf_scr[...]
    hx = lax.dot_general(xx, w1x_ref[0].astype(jnp.bfloat16),
                         (((1,), (1,)), ((), ())),
                         preferred_element_type=jnp.float32)
    hg = lax.dot_general(xx, w1g_ref[0].astype(jnp.bfloat16),
                         (((1,), (1,)), ((), ())),
                         preferred_element_type=jnp.float32)
    act = (0.5 * hg * (1.0 + lax.erf(hg * 0.7071067811865476))) * hx
    rows0 = e * _TPE
    gcols = gates_ref[pl.ds((e0 + e) * _TPE, _TPE), :]
    acomb = act[:_TPE] * gcols[:, 0:1] + act[_TPE:] * gcols[:, 1:2]
    contrib = lax.dot_general(acomb.astype(jnp.bfloat16),
                              w2_ref[0].astype(jnp.bfloat16),
                              (((1,), (1,)), ((), ())),
                              preferred_element_type=jnp.float32)

    @pl.when(ht == 0)
    def _init():
        out_ref[pl.ds(rows0, _TPE), :] = contrib

    @pl.when(ht != 0)
    def _acc():
        out_ref[pl.ds(rows0, _TPE), :] += contrib


def _expert_call(xs, W1, W2, gates, e0, ne):
    nt = ne * _TPE
    return pl.pallas_call(
        functools.partial(_expert_body, e0),
        grid=(ne, _NHT),
        in_specs=[
            pl.BlockSpec((1, _TPE, _D), lambda e, ht: (0, e, 0)),
            pl.BlockSpec((1, _TPE, _D), lambda e, ht: (1, e, 0)),
            pl.BlockSpec((1, _HC, _D), lambda e, ht: (e0 + e, ht, 0)),
            pl.BlockSpec((1, _HC, _D), lambda e, ht: (e0 + e, _NHT + ht, 0)),
            pl.BlockSpec((1, _D, _HC), lambda e, ht: (e0 + e, 0, ht)),
            pl.BlockSpec((_N, 2), lambda e, ht: (0, 0)),
        ],
        out_specs=pl.BlockSpec((nt, _D), lambda e, ht: (0, 0)),
        out_shape=jax.ShapeDtypeStruct((nt, _D), jnp.float32),
        scratch_shapes=[pltpu.VMEM((2 * _TPE, _D), jnp.bfloat16)],
    )(xs, xs, W1, W1, W2, gates)


_NSL = 1
_EPS = _E // _NSL
_TSL = _EPS * _TPE


@jax.jit
def kernel(inputs, W1, W2, Wr):
    x = inputs.reshape(_N, _D)
    probs, gidx, gates = _router_call(x, Wr)
    gidxT = gidx.T
    outs = []
    for sl in range(_NSL):
        gsl = gidxT[:, sl * _TSL:(sl + 1) * _TSL].reshape(2 * _TSL)
        xsl = _sc_gather(x, gsl).reshape(2, _TSL, _D)
        outs.append(_expert_call(xsl, W1, W2, gates, sl * _EPS, _EPS))
    out = jnp.concatenate(outs, axis=0)
    return out.reshape(_B, _S, _D), probs.reshape(_B, _S, _E)


# speedup vs baseline: 1.1515x; 1.0860x over previous
import functools
import jax
import jax.numpy as jnp
from jax import lax
from jax.experimental import pallas as pl
from jax.experimental.pallas import tpu as pltpu
from jax.experimental.pallas import tpu_sc as plsc

_B, _S = 2, 2048
_D = 1024
_H = 2048
_E = 8
_N = _B * _S
_TT = 256
_TPE = _N // _E
_NI = _TPE // _TT

_NC, _NS = 2, 16
_NW = _NC * _NS
_RPW = 2 * _N // _NW
_CH = 32
_NCH = _RPW // _CH


def _router_body(x_ref, wr_ref, probs_ref, gidx_ref, gates_ref):
    x = x_ref[...]
    logits = lax.dot_general(x, wr_ref[...], (((1,), (1,)), ((), ())),
                             preferred_element_type=jnp.float32)
    p1 = jax.nn.softmax(logits, axis=-1)
    probs = jax.nn.softmax(p1, axis=-1)
    lane = lax.broadcasted_iota(jnp.int32, (_N, _E), 1)
    m1 = jnp.max(probs, axis=-1, keepdims=True)
    i1 = jnp.min(jnp.where(probs == m1, lane, _E), axis=-1, keepdims=True)
    p_wo = jnp.where(lane == i1, -1.0, probs)
    m2 = jnp.max(p_wo, axis=-1, keepdims=True)
    i2 = jnp.min(jnp.where(p_wo == m2, lane, _E), axis=-1, keepdims=True)
    row = lax.broadcasted_iota(jnp.int32, (_N, 1), 0)
    base8 = 8 * (row % _TPE)
    probs_ref[...] = probs
    gidx_ref[:, 0:1] = base8 + i1
    gidx_ref[:, 1:2] = base8 + i2
    gates_ref[:, 0:1] = m1
    gates_ref[:, 1:2] = m2


def _router_call(x, Wr):
    return pl.pallas_call(
        _router_body,
        in_specs=[
            pl.BlockSpec((_N, _D), lambda: (0, 0)),
            pl.BlockSpec((_E, _D), lambda: (0, 0)),
        ],
        out_specs=[
            pl.BlockSpec((_N, _E), lambda: (0, 0)),
            pl.BlockSpec((_N, 2), lambda: (0, 0)),
            pl.BlockSpec((_N, 2), lambda: (0, 0)),
        ],
        out_shape=[
            jax.ShapeDtypeStruct((_N, _E), jnp.float32),
            jax.ShapeDtypeStruct((_N, 2), jnp.int32),
            jax.ShapeDtypeStruct((_N, 2), jnp.float32),
        ],
    )(x, Wr)


def _sc_gather(x, gidx_flat):
    mesh = plsc.VectorSubcoreMesh(core_axis_name="c", subcore_axis_name="s")
    row_w = x.shape[1]
    n_rows = gidx_flat.shape[0]
    rpw = n_rows // _NW
    nch = rpw // _CH

    @functools.partial(
        pl.kernel,
        mesh=mesh,
        out_type=jax.ShapeDtypeStruct((n_rows, row_w), x.dtype),
        scratch_types=[
            pltpu.VMEM((nch, _CH), jnp.int32),
            pltpu.VMEM((_CH, row_w), x.dtype),
            pltpu.VMEM((_CH, row_w), x.dtype),
            pltpu.SemaphoreType.DMA,
            pltpu.SemaphoreType.DMA,
        ],
    )
    def k(x_hbm, gidx_hbm, xs_hbm, idx_v, rows_a, rows_b, sem_a, sem_b):
        wid = lax.axis_index("s") * _NC + lax.axis_index("c")
        base = wid * rpw
        pltpu.sync_copy(gidx_hbm.at[wid], idx_v)
        bufs = (rows_a, rows_b)
        sems = (sem_a, sem_b)
        handles = [None, None]
        for c in range(nch):
            handles[c % 2] = pltpu.async_copy(
                x_hbm.at[idx_v.at[c]], bufs[c % 2], sems[c % 2])
            if c >= 1:
                handles[(c - 1) % 2].wait()
                pltpu.sync_copy(bufs[(c - 1) % 2],
                                xs_hbm.at[pl.ds(base + (c - 1) * _CH, _CH)])
        handles[(nch - 1) % 2].wait()
        pltpu.sync_copy(bufs[(nch - 1) % 2],
                        xs_hbm.at[pl.ds(base + (nch - 1) * _CH, _CH)])

    return k(x, gidx_flat.reshape(_NW, nch, _CH))


_HC = 1024
_NHT = _H // _HC


def _expert_body(e0, xs0_ref, xs1_ref, w1x_ref, w1g_ref, w2_ref, gates_ref,
                 out_ref, xbf_scr):
    e = pl.program_id(0)
    ht = pl.program_id(1)

    @pl.when(ht == 0)
    def _stage_x():
        xbf_scr[:_TPE, :] = xs0_ref[0].astype(jnp.bfloat16)
        xbf_scr[_TPE:, :] = xs1_ref[0].astype(jnp.bfloat16)

    xx = xbf_scr[...]
    hx = lax.dot_general(xx, w1x_ref[0].astype(jnp.bfloat16),
                         (((1,), (1,)), ((), ())),
                         preferred_element_type=jnp.float32)
    hg = lax.dot_general(xx, w1g_ref[0].astype(jnp.bfloat16),
                         (((1,), (1,)), ((), ())),
                         preferred_element_type=jnp.float32)
    act = (0.5 * hg * (1.0 + lax.erf(hg * 0.7071067811865476))) * hx
    rows0 = e * _TPE
    gcols = gates_ref[pl.ds((e0 + e) * _TPE, _TPE), :]
    acomb = act[:_TPE] * gcols[:, 0:1] + act[_TPE:] * gcols[:, 1:2]
    contrib = lax.dot_general(acomb.astype(jnp.bfloat16),
                              w2_ref[0].astype(jnp.bfloat16),
                              (((1,), (1,)), ((), ())),
                              preferred_element_type=jnp.float32)

    del rows0

    @pl.when(ht == 0)
    def _init():
        out_ref[...] = contrib

    @pl.when(ht != 0)
    def _acc():
        out_ref[...] += contrib


def _expert_call(xs, W1, W2, gates, e0, ne):
    nt = ne * _TPE
    return pl.pallas_call(
        functools.partial(_expert_body, e0),
        grid=(ne, _NHT),
        in_specs=[
            pl.BlockSpec((1, _TPE, _D), lambda e, ht: (0, e, 0)),
            pl.BlockSpec((1, _TPE, _D), lambda e, ht: (1, e, 0)),
            pl.BlockSpec((1, _HC, _D), lambda e, ht: (e0 + e, ht, 0)),
            pl.BlockSpec((1, _HC, _D), lambda e, ht: (e0 + e, _NHT + ht, 0)),
            pl.BlockSpec((1, _D, _HC), lambda e, ht: (e0 + e, 0, ht)),
            pl.BlockSpec((_N, 2), lambda e, ht: (0, 0)),
        ],
        out_specs=pl.BlockSpec((_TPE, _D), lambda e, ht: (e, 0)),
        out_shape=jax.ShapeDtypeStruct((nt, _D), jnp.float32),
        scratch_shapes=[pltpu.VMEM((2 * _TPE, _D), jnp.bfloat16)],
    )(xs, xs, W1, W1, W2, gates)


_NSL = 1
_EPS = _E // _NSL
_TSL = _EPS * _TPE


@jax.jit
def kernel(inputs, W1, W2, Wr):
    x = inputs.reshape(_N, _D)
    probs, gidx, gates = _router_call(x, Wr)
    gidxT = gidx.T
    outs = []
    for sl in range(_NSL):
        gsl = gidxT[:, sl * _TSL:(sl + 1) * _TSL].reshape(2 * _TSL)
        xsl = _sc_gather(x, gsl).reshape(2, _TSL, _D)
        outs.append(_expert_call(xsl, W1, W2, gates, sl * _EPS, _EPS))
    out = jnp.concatenate(outs, axis=0)
    return out.reshape(_B, _S, _D), probs.reshape(_B, _S, _E)


# speedup vs baseline: 1.1665x; 1.0130x over previous
import functools
import jax
import jax.numpy as jnp
from jax import lax
from jax.experimental import pallas as pl
from jax.experimental.pallas import tpu as pltpu
from jax.experimental.pallas import tpu_sc as plsc

_B, _S = 2, 2048
_D = 1024
_H = 2048
_E = 8
_N = _B * _S
_TT = 256
_TPE = _N // _E
_NI = _TPE // _TT

_NC, _NS = 2, 16
_NW = _NC * _NS
_RPW = 2 * _N // _NW
_CH = 32
_NCH = _RPW // _CH


def _router_body(x_ref, wr_ref, probs_ref, gidx_ref, gates_ref):
    x = x_ref[...]
    logits = lax.dot_general(x, wr_ref[...], (((1,), (1,)), ((), ())),
                             preferred_element_type=jnp.float32)
    p1 = jax.nn.softmax(logits, axis=-1)
    probs = jax.nn.softmax(p1, axis=-1)
    lane = lax.broadcasted_iota(jnp.int32, (_N, _E), 1)
    m1 = jnp.max(probs, axis=-1, keepdims=True)
    i1 = jnp.min(jnp.where(probs == m1, lane, _E), axis=-1, keepdims=True)
    p_wo = jnp.where(lane == i1, -1.0, probs)
    m2 = jnp.max(p_wo, axis=-1, keepdims=True)
    i2 = jnp.min(jnp.where(p_wo == m2, lane, _E), axis=-1, keepdims=True)
    row = lax.broadcasted_iota(jnp.int32, (_N, 1), 0)
    base8 = 8 * (row % _TPE)
    probs_ref[...] = probs
    gidx_ref[:, 0:1] = base8 + i1
    gidx_ref[:, 1:2] = base8 + i2
    gates_ref[:, 0:1] = m1
    gates_ref[:, 1:2] = m2


def _router_call(x, Wr):
    return pl.pallas_call(
        _router_body,
        in_specs=[
            pl.BlockSpec((_N, _D), lambda: (0, 0)),
            pl.BlockSpec((_E, _D), lambda: (0, 0)),
        ],
        out_specs=[
            pl.BlockSpec((_N, _E), lambda: (0, 0)),
            pl.BlockSpec((_N, 2), lambda: (0, 0)),
            pl.BlockSpec((_N, 2), lambda: (0, 0)),
        ],
        out_shape=[
            jax.ShapeDtypeStruct((_N, _E), jnp.float32),
            jax.ShapeDtypeStruct((_N, 2), jnp.int32),
            jax.ShapeDtypeStruct((_N, 2), jnp.float32),
        ],
    )(x, Wr)


def _sc_gather(x, gidx_flat):
    mesh = plsc.VectorSubcoreMesh(core_axis_name="c", subcore_axis_name="s")
    row_w = x.shape[1]
    n_rows = gidx_flat.shape[0]
    rpw = n_rows // _NW
    nch = rpw // _CH

    @functools.partial(
        pl.kernel,
        mesh=mesh,
        out_type=jax.ShapeDtypeStruct((n_rows, row_w), x.dtype),
        scratch_types=[
            pltpu.VMEM((nch, _CH), jnp.int32),
            pltpu.VMEM((_CH, row_w), x.dtype),
            pltpu.VMEM((_CH, row_w), x.dtype),
            pltpu.SemaphoreType.DMA,
            pltpu.SemaphoreType.DMA,
        ],
    )
    def k(x_hbm, gidx_hbm, xs_hbm, idx_v, rows_a, rows_b, sem_a, sem_b):
        wid = lax.axis_index("s") * _NC + lax.axis_index("c")
        base = wid * rpw
        pltpu.sync_copy(gidx_hbm.at[wid], idx_v)
        bufs = (rows_a, rows_b)
        sems = (sem_a, sem_b)
        handles = [None, None]
        for c in range(nch):
            handles[c % 2] = pltpu.async_copy(
                x_hbm.at[idx_v.at[c]], bufs[c % 2], sems[c % 2])
            if c >= 1:
                handles[(c - 1) % 2].wait()
                pltpu.sync_copy(bufs[(c - 1) % 2],
                                xs_hbm.at[pl.ds(base + (c - 1) * _CH, _CH)])
        handles[(nch - 1) % 2].wait()
        pltpu.sync_copy(bufs[(nch - 1) % 2],
                        xs_hbm.at[pl.ds(base + (nch - 1) * _CH, _CH)])

    return k(x, gidx_flat.reshape(_NW, nch, _CH))


_HC = 1024
_NHT = _H // _HC


def _expert_body(e0, xs0_ref, xs1_ref, w1x_ref, w1g_ref, w2_ref, gates_ref,
                 out_ref, xbf_scr):
    e = pl.program_id(0)
    ht = pl.program_id(1)

    @pl.when(ht == 0)
    def _stage_x():
        xbf_scr[:_TPE, :] = xs0_ref[0].astype(jnp.bfloat16)
        xbf_scr[_TPE:, :] = xs1_ref[0].astype(jnp.bfloat16)

    xx = xbf_scr[...]
    hx = lax.dot_general(xx, w1x_ref[0].astype(jnp.bfloat16),
                         (((1,), (1,)), ((), ())),
                         preferred_element_type=jnp.float32)
    hg = lax.dot_general(xx, w1g_ref[0].astype(jnp.bfloat16),
                         (((1,), (1,)), ((), ())),
                         preferred_element_type=jnp.float32)
    act = (0.5 * hg * (1.0 + lax.erf(hg * 0.7071067811865476))) * hx
    gcols = gates_ref[...]
    acomb = act[:_TPE] * gcols[:, 0:1] + act[_TPE:] * gcols[:, 1:2]
    contrib = lax.dot_general(acomb.astype(jnp.bfloat16),
                              w2_ref[0].astype(jnp.bfloat16),
                              (((1,), (1,)), ((), ())),
                              preferred_element_type=jnp.float32)

    @pl.when(ht == 0)
    def _init():
        out_ref[...] = contrib

    @pl.when(ht != 0)
    def _acc():
        out_ref[...] += contrib


def _expert_call(xs, W1, W2, gates, e0, ne):
    nt = ne * _TPE
    return pl.pallas_call(
        functools.partial(_expert_body, e0),
        grid=(ne, _NHT),
        in_specs=[
            pl.BlockSpec((1, _TPE, _D), lambda e, ht: (0, e, 0)),
            pl.BlockSpec((1, _TPE, _D), lambda e, ht: (1, e, 0)),
            pl.BlockSpec((1, _HC, _D), lambda e, ht: (e0 + e, ht, 0)),
            pl.BlockSpec((1, _HC, _D), lambda e, ht: (e0 + e, _NHT + ht, 0)),
            pl.BlockSpec((1, _D, _HC), lambda e, ht: (e0 + e, 0, ht)),
            pl.BlockSpec((_TPE, 2), lambda e, ht: (e0 + e, 0)),
        ],
        out_specs=pl.BlockSpec((_TPE, _D), lambda e, ht: (e, 0)),
        out_shape=jax.ShapeDtypeStruct((nt, _D), jnp.float32),
        scratch_shapes=[pltpu.VMEM((2 * _TPE, _D), jnp.bfloat16)],
    )(xs, xs, W1, W1, W2, gates)


_NSL = 1
_EPS = _E // _NSL
_TSL = _EPS * _TPE


@jax.jit
def kernel(inputs, W1, W2, Wr):
    x = inputs.reshape(_N, _D)
    probs, gidx, gates = _router_call(x, Wr)
    gidxT = gidx.T
    outs = []
    for sl in range(_NSL):
        gsl = gidxT[:, sl * _TSL:(sl + 1) * _TSL].reshape(2 * _TSL)
        xsl = _sc_gather(x, gsl).reshape(2, _TSL, _D)
        outs.append(_expert_call(xsl, W1, W2, gates, sl * _EPS, _EPS))
    out = jnp.concatenate(outs, axis=0)
    return out.reshape(_B, _S, _D), probs.reshape(_B, _S, _E)
